# Initial kernel scaffold; baseline (speedup 1.0000x reference)
import jax
import jax.numpy as jnp
from jax.experimental import pallas as pl


def kernel(x, W1, b1, W2, b2):
    raise NotImplementedError("write your pallas kernel here")


# speedup vs baseline: 1.4040x; 1.4040x over previous
import functools

import jax
import jax.numpy as jnp
from jax.experimental import pallas as pl
from jax.experimental.pallas import tpu as pltpu

L = 2048
Y = 2048
C = 32
H = 32
CH = C * H
TWO_PI = 6.283185307179586
T_CHUNK = 512


def _kernel(x_ref, w1f_ref, b1f_ref, w2f_ref, b2_ref,
            cx_ref, cy_ref, tx_ref, ty_ref, per_ref):
    b = pl.program_id(0)

    @pl.when(b == 0)
    def _compute_periodic():
        rowc = jax.lax.broadcasted_iota(jnp.int32, (CH, C), 0) // H
        colc = jax.lax.broadcasted_iota(jnp.int32, (CH, C), 1)
        msel = jnp.where(rowc == colc, w2f_ref[:, :], 0.0)
        w1s = w1f_ref[0, :][None, :]
        w1c = w1f_ref[1, :][None, :]
        b1f = b1f_ref[0, :][None, :]
        b2r = b2_ref[0, :][None, :]
        for k in range(L // T_CHUNK):
            i = jax.lax.broadcasted_iota(jnp.int32, (T_CHUNK, 1), 0) + k * T_CHUNK
            t = i.astype(jnp.float32) * (1.0 / L)
            phase = TWO_PI * t
            s = jnp.sin(phase)
            c = jnp.cos(phase)
            h = s * w1s + c * w1c + b1f
            h = jnp.maximum(h, 0.0)
            per = jnp.dot(h, msel, preferred_element_type=jnp.float32) + b2r
            per_ref[pl.ds(k * T_CHUNK, T_CHUNK), :] = per

    i = jax.lax.broadcasted_iota(jnp.int32, (1, L + Y), 1)
    marks = jnp.where(i < L,
                      i.astype(jnp.float32) * (1.0 / L),
                      (i - L).astype(jnp.float32) * (1.0 / Y))
    cx_ref[0, :, :] = marks
    tx_ref[0, :, :] = marks

    resid = x_ref[0, :, :] - per_ref[:, :]
    cy_ref[0, :L, :C] = resid
    cy_ref[0, :L, C:] = jnp.ones((L, C), jnp.float32)
    cy_ref[0, L:, :] = jnp.zeros((Y, 2 * C), jnp.float32)
    ty_ref[0, :L, :] = jnp.zeros((L, 2 * C), jnp.float32)
    ty_ref[0, L:, :] = jnp.ones((Y, 2 * C), jnp.float32)


@jax.jit
def kernel(x, W1, b1, W2, b2):
    B = x.shape[0]
    w1f = W1.transpose(1, 0, 2).reshape(2, CH)
    b1f = b1.reshape(1, CH)
    w2f = W2.reshape(CH, 1)
    b2r = b2.reshape(1, C)

    grid = (B,)
    out_shapes = (
        jax.ShapeDtypeStruct((B, 1, L + Y), jnp.float32),
        jax.ShapeDtypeStruct((B, L + Y, 2 * C), jnp.float32),
        jax.ShapeDtypeStruct((B, 1, L + Y), jnp.float32),
        jax.ShapeDtypeStruct((B, L + Y, 2 * C), jnp.float32),
    )
    in_specs = [
        pl.BlockSpec((1, L, C), lambda b: (b, 0, 0)),
        pl.BlockSpec((2, CH), lambda b: (0, 0)),
        pl.BlockSpec((1, CH), lambda b: (0, 0)),
        pl.BlockSpec((CH, 1), lambda b: (0, 0)),
        pl.BlockSpec((1, C), lambda b: (0, 0)),
    ]
    out_specs = (
        pl.BlockSpec((1, 1, L + Y), lambda b: (b, 0, 0)),
        pl.BlockSpec((1, L + Y, 2 * C), lambda b: (b, 0, 0)),
        pl.BlockSpec((1, 1, L + Y), lambda b: (b, 0, 0)),
        pl.BlockSpec((1, L + Y, 2 * C), lambda b: (b, 0, 0)),
    )
    cx, cy, tx, ty = pl.pallas_call(
        _kernel,
        grid=grid,
        in_specs=in_specs,
        out_specs=out_specs,
        out_shape=out_shapes,
        scratch_shapes=[pltpu.VMEM((L, C), jnp.float32)],
    )(x, w1f, b1f, w2f, b2r)
    return (cx.reshape(B, L + Y), cy, tx.reshape(B, L + Y), ty)
